# baseline (speedup 1.0000x reference)
import math

import jax
import jax.numpy as jnp
from jax import lax
from jax.experimental import pallas as pl
from jax.experimental.pallas import tpu as pltpu
from jax.experimental.pallas import tpu_sc as plsc

D = 64
SCALE = math.sqrt(D)
V = 1000000
B = 16384
S = 50

NUM_CORES = 2
NUM_SUBCORES = 16
NW = NUM_CORES * NUM_SUBCORES
LANES = 16

BSTRIPE = B // NW
GCHUNK = 128


HSTRIPE = BSTRIPE // 2


def _body(xt_hbm, emb_hbm, out_hbm, idx_v, g_v, o_v,
          sem, sem2, semw, semw2):
    wid = lax.axis_index("s") * NUM_CORES + lax.axis_index("c")
    b0 = wid * BSTRIPE
    pltpu.sync_copy(xt_hbm.at[:, pl.ds(b0, BSTRIPE)], idx_v)
    gsems = (sem, sem2)
    wsems = (semw, semw2)

    def gather_unit(s, half, gbuf):
        return [
            pltpu.async_copy(
                emb_hbm.at[idx_v.at[
                    s, pl.ds(half * HSTRIPE + h * GCHUNK, GCHUNK)]],
                g_v.at[gbuf, pl.ds(h * GCHUNK, GCHUNK)],
                gsems[gbuf],
            )
            for h in range(HSTRIPE // GCHUNK)
        ]

    def transpose_unit(buf, s, half, first):
        boff = b0 + half * HSTRIPE
        dst = out_hbm.at[s, :, pl.ds(boff, HSTRIPE)]
        osrc = o_v.at[buf, :, pl.ds(0, HSTRIPE)]

        @pl.when(jnp.logical_not(first))
        def _():
            pltpu.make_async_copy(osrc, dst, wsems[buf]).wait()

        ob = o_v.at[buf]

        def row_body(j, c2):
            for cg in range(D // LANES):
                vals = g_v[buf, j, pl.ds(cg * LANES, LANES)] * SCALE
                cvec = lax.iota(jnp.int32, LANES) + cg * LANES
                jvec = jnp.full((LANES,), j, jnp.int32)
                plsc.store_scatter(ob, [cvec, jvec], vals)
            return c2

        lax.fori_loop(0, HSTRIPE, row_body, 0, unroll=16)
        pltpu.async_copy(osrc, dst, wsems[buf])

    for cp in gather_unit(0, 0, 0):
        cp.wait()

    def t_body(tt, c3):
        cps = gather_unit(tt, 1, 1)
        transpose_unit(0, tt, 0, tt == 0)
        for cp in cps:
            cp.wait()

        @pl.when(tt + 1 < S)
        def _():
            cps2 = gather_unit(tt + 1, 0, 0)
            transpose_unit(1, tt, 1, tt == 0)
            for cp in cps2:
                cp.wait()

        @pl.when(tt + 1 == S)
        def _():
            transpose_unit(1, tt, 1, tt == 0)

        return c3

    lax.fori_loop(0, S, t_body, 0)
    pltpu.make_async_copy(
        o_v.at[0, :, pl.ds(0, HSTRIPE)],
        out_hbm.at[S - 1, :, pl.ds(b0, HSTRIPE)],
        semw,
    ).wait()
    pltpu.make_async_copy(
        o_v.at[1, :, pl.ds(0, HSTRIPE)],
        out_hbm.at[S - 1, :, pl.ds(b0 + HSTRIPE, HSTRIPE)],
        semw2,
    ).wait()


@jax.jit
def _lookup(xt, emb):
    mesh = plsc.VectorSubcoreMesh(core_axis_name="c", subcore_axis_name="s")
    f = pl.kernel(
        _body,
        out_type=jax.ShapeDtypeStruct((S, D, B), jnp.float32),
        mesh=mesh,
        scratch_types=[
            pltpu.VMEM((S, BSTRIPE), jnp.int32),
            pltpu.VMEM((2, HSTRIPE, D), jnp.float32),
            pltpu.VMEM((2, D, HSTRIPE + 1), jnp.float32),
            pltpu.SemaphoreType.DMA,
            pltpu.SemaphoreType.DMA,
            pltpu.SemaphoreType.DMA,
            pltpu.SemaphoreType.DMA,
        ],
        compiler_params=pltpu.CompilerParams(
            use_tc_tiling_on_sc=False, needs_layout_passes=False),
    )
    return f(xt, emb)


def kernel(x, emb):
    xt = x.T.astype(jnp.int32)
    out3 = _lookup(xt, emb)
    return out3.transpose(2, 0, 1)
